# Initial kernel scaffold; baseline (speedup 1.0000x reference)
import jax
import jax.numpy as jnp
from jax.experimental import pallas as pl


def kernel(A, B, prototypes):
    raise NotImplementedError("write your pallas kernel here")


# speedup vs baseline: 28.7534x; 28.7534x over previous
import functools

import jax
import jax.numpy as jnp
from jax import lax
from jax.experimental import pallas as pl
from jax.experimental.pallas import tpu as pltpu

N, D, M, C, K = 2048, 1024, 512, 64, 16
CK = C * K
NT = 256
GRID = N // NT


def _lut_norms_body(p_ref, b_ref, lut_ref, norms_ref):
    p = p_ref[...]
    lut_ref[...] = lax.dot_general(
        p, b_ref[...], (((1,), (0,)), ((), ())),
        preferred_element_type=jnp.float32,
        precision=lax.Precision.DEFAULT)
    norms_ref[...] = jnp.sum(p * p, axis=1, keepdims=True)


def _encode_agg_body(p_ref, at_ref, norms_ref, lut_ref, out_ref):
    scores = lax.dot_general(
        p_ref[...], at_ref[...], (((1,), (0,)), ((), ())),
        preferred_element_type=jnp.float32,
        precision=lax.Precision.DEFAULT)
    fit = 2.0 * scores - norms_ref[...]
    fit3 = fit.reshape(C, K, NT)
    maxv = jnp.max(fit3, axis=1, keepdims=True)
    kio = lax.broadcasted_iota(jnp.int32, (C, K, NT), 1)
    ksel = jnp.min(jnp.where(fit3 == maxv, kio, K), axis=1)
    oh = (kio == ksel[:, None, :]).astype(jnp.float32).reshape(CK, NT)
    out_ref[...] = lax.dot_general(
        lut_ref[...], oh, (((0,), (0,)), ((), ())),
        preferred_element_type=jnp.float32,
        precision=lax.Precision.HIGHEST)


@jax.jit
def kernel(A, B, prototypes):
    P = prototypes.reshape(CK, D)
    lut, norms = pl.pallas_call(
        _lut_norms_body,
        out_shape=(jax.ShapeDtypeStruct((CK, M), jnp.float32),
                   jax.ShapeDtypeStruct((CK, 1), jnp.float32)),
    )(P, B)
    A_T = A.T
    out_T = pl.pallas_call(
        _encode_agg_body,
        grid=(GRID,),
        in_specs=[
            pl.BlockSpec((CK, D), lambda i: (0, 0)),
            pl.BlockSpec((D, NT), lambda i: (0, i)),
            pl.BlockSpec((CK, 1), lambda i: (0, 0)),
            pl.BlockSpec((CK, M), lambda i: (0, 0)),
        ],
        out_specs=pl.BlockSpec((M, NT), lambda i: (0, i)),
        out_shape=jax.ShapeDtypeStruct((M, N), jnp.float32),
    )(P, A_T, norms, lut)
    return out_T.T
